# Initial kernel scaffold; baseline (speedup 1.0000x reference)
import jax
import jax.numpy as jnp
from jax.experimental import pallas as pl


def kernel(x, edge_index, W1, b1, g1, be1, a1, W2, b2, g2, be2, a2, Wm1, bm1, Wm2, bm2, Wm3, bm3):
    raise NotImplementedError("write your pallas kernel here")


# speedup vs baseline: 13.2243x; 13.2243x over previous
import functools

import jax
import jax.numpy as jnp
from jax import lax
from jax.experimental import pallas as pl
from jax.experimental.pallas import tpu as pltpu
from jax.experimental.pallas import tpu_sc as plsc

NC = 2
NS = 16
NW = NC * NS
CH = 128
FEAT = 128

N_NODES = 10000
N_P = 10112
TRASH = N_NODES
DEG_PAD = 10240
ROWS_PER_TILE = N_P // NS


@functools.cache
def _sc_degree(k_chunks):
    mesh = plsc.VectorSubcoreMesh(core_axis_name="c", subcore_axis_name="s")

    def body(z_hbm, ones_hbm, dst_hbm, deg_out, onesb, dstt, acc):
        c = lax.axis_index("c")
        s = lax.axis_index("s")
        w = c * NS + s

        @pl.when(s == 0)
        def _():
            pltpu.sync_copy(z_hbm, acc)
        pltpu.sync_copy(ones_hbm, onesb)
        pltpu.sync_copy(dst_hbm.at[w], dstt)
        plsc.subcore_barrier()

        def step(j, carry):
            pltpu.sync_copy(onesb, acc.at[dstt.at[j]], add=True)
            return carry

        lax.fori_loop(0, k_chunks, step, 0)
        plsc.subcore_barrier()

        @pl.when(s == 0)
        def _():
            pltpu.sync_copy(acc, deg_out.at[c])

    return pl.kernel(
        body,
        out_type=jax.ShapeDtypeStruct((NC, DEG_PAD), jnp.float32),
        mesh=mesh,
        scratch_types=[
            pltpu.VMEM((CH,), jnp.float32),
            pltpu.VMEM((k_chunks, CH), jnp.int32),
            pltpu.VMEM_SHARED((DEG_PAD,), jnp.float32),
        ],
    )


@functools.cache
def _sc_edges(k_chunks):
    mesh = plsc.VectorSubcoreMesh(core_axis_name="c", subcore_axis_name="s")

    def body(hp_hbm, src_hbm, dst_hbm, out_hbm, srct, dstt, msg, acc):
        c = lax.axis_index("c")
        s = lax.axis_index("s")
        w = c * NS + s
        row0 = pl.multiple_of(s * ROWS_PER_TILE, 8)

        pltpu.sync_copy(hp_hbm.at[pl.ds(row0, ROWS_PER_TILE)],
                        acc.at[pl.ds(row0, ROWS_PER_TILE)])
        pltpu.sync_copy(src_hbm.at[w], srct)
        pltpu.sync_copy(dst_hbm.at[w], dstt)
        plsc.subcore_barrier()

        def step(j, carry):
            pltpu.sync_copy(hp_hbm.at[srct.at[j]], msg)
            pltpu.sync_copy(msg, acc.at[dstt.at[j]], add=True)
            return carry

        lax.fori_loop(0, k_chunks, step, 0)
        plsc.subcore_barrier()

        pltpu.sync_copy(acc.at[pl.ds(row0, ROWS_PER_TILE)],
                        out_hbm.at[c].at[pl.ds(row0, ROWS_PER_TILE)])

    return pl.kernel(
        body,
        out_type=jax.ShapeDtypeStruct((NC, N_P, FEAT), jnp.float32),
        mesh=mesh,
        scratch_types=[
            pltpu.VMEM((k_chunks, CH), jnp.int32),
            pltpu.VMEM((k_chunks, CH), jnp.int32),
            pltpu.VMEM((CH, FEAT), jnp.float32),
            pltpu.VMEM_SHARED((N_P, FEAT), jnp.float32),
        ],
    )


def _valid_mask(shape):
    rows = lax.broadcasted_iota(jnp.int32, shape, 0)
    return rows < N_NODES


def _tc_pre_body(degt_ref, x_ref, w1_ref, hp_ref, dinv_ref):
    deg = degt_ref[0] + degt_ref[1] + 1.0
    dinv = lax.rsqrt(deg)
    dinv_ref[...] = dinv
    h = jnp.dot(x_ref[...], w1_ref[...], preferred_element_type=jnp.float32)
    hp_ref[...] = h * dinv


def _graph_norm_leaky(z, g, be, a):
    valid = _valid_mask(z.shape)
    zm = jnp.where(valid, z, 0.0)
    mean = jnp.sum(zm, axis=0, keepdims=True) / N_NODES
    xc = z - a * mean
    xcm = jnp.where(valid, xc, 0.0)
    var = jnp.sum(xcm * xcm, axis=0, keepdims=True) / N_NODES
    y = g * xc * lax.rsqrt(var + 1e-5) + be
    y = jnp.where(y >= 0, y, 0.01 * y)
    return jnp.where(valid, y, 0.0)


def _tc_mid_body(dinv_ref, p_ref, hp_ref, b_ref, g_ref, be_ref, a_ref, w2_ref,
                 hp2_ref):
    dinv = dinv_ref[...]
    z = (p_ref[0] + p_ref[1] - hp_ref[...]) * dinv + b_ref[...]
    y = _graph_norm_leaky(z, g_ref[...], be_ref[...], a_ref[...])
    hp2_ref[...] = jnp.dot(y, w2_ref[...],
                           preferred_element_type=jnp.float32) * dinv


def _tc_post_body(dinv_ref, p_ref, hp_ref, b_ref, g_ref, be_ref, a_ref,
                  wm1_ref, bm1_ref, wm2_ref, bm2_ref, wm3_ref, bm3_ref,
                  out_ref):
    dinv = dinv_ref[...]
    z = (p_ref[0] + p_ref[1] - hp_ref[...]) * dinv + b_ref[...]
    y = _graph_norm_leaky(z, g_ref[...], be_ref[...], a_ref[...])
    h1 = jnp.dot(y, wm1_ref[...], preferred_element_type=jnp.float32)
    h1 = jnp.maximum(h1 + bm1_ref[...], 0.0)
    h2 = jnp.dot(h1, wm2_ref[...], preferred_element_type=jnp.float32)
    h2 = jnp.maximum(h2 + bm2_ref[...], 0.0)
    h3 = jnp.dot(h2, wm3_ref[...], preferred_element_type=jnp.float32)
    out_ref[...] = (h3 + bm3_ref[...])[:N_NODES]


def _pad2(m, rows, cols):
    return jnp.pad(m, ((0, rows - m.shape[0]), (0, cols - m.shape[1])))


def _padrow(v, cols):
    return jnp.pad(v.reshape(1, -1), ((0, 0), (0, cols - v.shape[0])))


def kernel(x, edge_index, W1, b1, g1, be1, a1, W2, b2, g2, be2, a2,
           Wm1, bm1, Wm2, bm2, Wm3, bm3):
    n, d_in = x.shape
    e = edge_index.shape[1]
    k_chunks = -(-e // (NW * CH))
    e_pad = NW * k_chunks * CH

    src = edge_index[0]
    dst = edge_index[1]
    pad = e_pad - e
    src_p = jnp.concatenate(
        [src, jnp.zeros((pad,), jnp.int32)]).reshape(NW, k_chunks, CH)
    dst_p = jnp.concatenate(
        [dst, jnp.full((pad,), TRASH, jnp.int32)]).reshape(NW, k_chunks, CH)
    xp = jnp.pad(x, ((0, N_P - n), (0, 0)))

    deg_parts = _sc_degree(k_chunks)(
        jnp.zeros((DEG_PAD,), jnp.float32), jnp.ones((CH,), jnp.float32),
        dst_p)
    degt = deg_parts[:, :N_P].reshape(NC, N_P, 1)

    w1p = _pad2(W1, d_in, FEAT)
    w2p = _pad2(W2, FEAT, FEAT)
    wm1p = _pad2(Wm1, FEAT, Wm1.shape[1])

    hp1, dinv = pl.pallas_call(
        _tc_pre_body,
        out_shape=(jax.ShapeDtypeStruct((N_P, FEAT), jnp.float32),
                   jax.ShapeDtypeStruct((N_P, 1), jnp.float32)),
    )(degt, xp, w1p)

    p1 = _sc_edges(k_chunks)(hp1, src_p, dst_p)

    hp2 = pl.pallas_call(
        _tc_mid_body,
        out_shape=jax.ShapeDtypeStruct((N_P, FEAT), jnp.float32),
    )(dinv, p1, hp1, _padrow(b1, FEAT), _padrow(g1, FEAT),
      _padrow(be1, FEAT), _padrow(a1, FEAT), w2p)

    p2 = _sc_edges(k_chunks)(hp2, src_p, dst_p)

    out = pl.pallas_call(
        _tc_post_body,
        out_shape=jax.ShapeDtypeStruct((n, 1), jnp.float32),
    )(dinv, p2, hp2, _padrow(b2, FEAT), _padrow(g2, FEAT),
      _padrow(be2, FEAT), _padrow(a2, FEAT),
      wm1p, bm1.reshape(1, -1), Wm2, bm2.reshape(1, -1),
      Wm3, bm3.reshape(1, -1))

    return out
